# baseline (speedup 1.0000x reference)
import functools

import jax
import jax.numpy as jnp
from jax import lax
from jax.experimental import pallas as pl
from jax.experimental.pallas import tpu as pltpu
from jax.experimental.pallas import tpu_sc as plsc

NC, NS, L = 2, 16, 16
NW = NC * NS
BATCH, INPUTS, OUTPUTS = 1024, 16384, 32768
ROWS_PER_W = BATCH // NW
PAIRS = ROWS_PER_W // 2
CH = 8192
NCH = OUTPUTS // CH
CWORDS = CH // 2


def _gather_body(x_hbm, idx_hbm, out_hbm, idx_v, pair0_v, pair1_v,
                 outa_v, outb_v, idx_sem, in_sem0, in_sem1,
                 out_sem0, out_sem1):
    wid = lax.axis_index("s") * NC + lax.axis_index("c")
    base = wid * ROWS_PER_W

    pairs = (pair0_v, pair1_v)
    outs = (outa_v, outb_v)
    in_sems = (in_sem0, in_sem1)
    out_sems = (out_sem0, out_sem1)
    in_copies = [None, None]
    out_copies = [None, None]

    idx_copy = pltpu.async_copy(idx_hbm, idx_v, idx_sem)
    in_copies[0] = pltpu.async_copy(
        x_hbm.at[pl.ds(base * INPUTS, 2 * INPUTS)], pairs[0], in_sems[0])
    idx_copy.wait()
    t = 0
    for p in range(PAIRS):
        cur = p & 1
        if p + 1 < PAIRS:
            in_copies[1 - cur] = pltpu.async_copy(
                x_hbm.at[pl.ds((base + 2 * (p + 1)) * INPUTS, 2 * INPUTS)],
                pairs[1 - cur], in_sems[1 - cur])
        in_copies[cur].wait()
        pair_ref = pairs[cur]
        for c in range(NCH):
            b = t & 1
            t += 1
            if out_copies[b] is not None:
                out_copies[b].wait()
            out_ref = outs[b]

            @plsc.parallel_loop(0, CWORDS, step=L, unroll=4)
            def _chunk(w, c=c, pair_ref=pair_ref, out_ref=out_ref):
                v = idx_v[pl.ds(c * CWORDS + w, L)]
                lo = v & 0xFFFF
                hi = v >> 16
                lo1 = lo + INPUTS
                hi1 = hi + INPUTS
                out_ref[0, pl.ds(2 * w, L)] = plsc.load_gather(pair_ref, [lo])
                out_ref[0, pl.ds(2 * w + L, L)] = plsc.load_gather(pair_ref, [hi])
                out_ref[1, pl.ds(2 * w, L)] = plsc.load_gather(pair_ref, [lo1])
                out_ref[1, pl.ds(2 * w + L, L)] = plsc.load_gather(pair_ref, [hi1])

            out_copies[b] = pltpu.async_copy(
                out_ref,
                out_hbm.at[pl.ds(base + 2 * p, 2), pl.ds(c * CH, CH)],
                out_sems[b])
    for b in range(2):
        if out_copies[b] is not None:
            out_copies[b].wait()


_gather_call = functools.partial(
    pl.kernel,
    out_type=jax.ShapeDtypeStruct((BATCH, OUTPUTS), jnp.float32),
    mesh=plsc.VectorSubcoreMesh(
        core_axis_name="c", subcore_axis_name="s",
        num_cores=NC, num_subcores=NS,
    ),
    scratch_types=[
        pltpu.VMEM((OUTPUTS // 2,), jnp.int32),
        pltpu.VMEM((2 * INPUTS,), jnp.float32),
        pltpu.VMEM((2 * INPUTS,), jnp.float32),
        pltpu.VMEM((2, CH), jnp.float32),
        pltpu.VMEM((2, CH), jnp.float32),
        pltpu.SemaphoreType.DMA,
        pltpu.SemaphoreType.DMA,
        pltpu.SemaphoreType.DMA,
        pltpu.SemaphoreType.DMA,
        pltpu.SemaphoreType.DMA,
    ],
    compiler_params=pltpu.CompilerParams(needs_layout_passes=False),
)(_gather_body)


def kernel(x, indices):
    u = indices.astype(jnp.uint32)
    blk = u.reshape(-1, 2, L)
    packed = blk[:, 0, :] | (blk[:, 1, :] << 16)
    idx_words = packed.reshape(-1).astype(jnp.int32)
    return _gather_call(x.reshape(-1), idx_words)


# speedup vs baseline: 1.4796x; 1.4796x over previous
import functools

import jax
import jax.numpy as jnp
from jax import lax
from jax.experimental import pallas as pl
from jax.experimental.pallas import tpu as pltpu
from jax.experimental.pallas import tpu_sc as plsc

NC, NS, L = 2, 16, 16
NW = NC * NS
BATCH, INPUTS, OUTPUTS = 1024, 16384, 32768
ROWS_PER_W = BATCH // NW
WORDS = OUTPUTS // 2


def _gather_body(x_hbm, idx_hbm, out_hbm, idx_v, row0_v, row1_v,
                 outa_v, outb_v, idx_sem, in_sem0, in_sem1,
                 out_sem0, out_sem1):
    wid = lax.axis_index("s") * NC + lax.axis_index("c")
    base = wid * ROWS_PER_W

    rows = (row0_v, row1_v)
    outs = (outa_v, outb_v)
    in_sems = (in_sem0, in_sem1)
    out_sems = (out_sem0, out_sem1)
    in_copies = [None, None]
    out_copies = [None, None]

    idx_copy = pltpu.async_copy(idx_hbm, idx_v, idx_sem)
    in_copies[0] = pltpu.async_copy(x_hbm.at[base], rows[0], in_sems[0])
    idx_copy.wait()
    for r in range(ROWS_PER_W):
        cur = r & 1
        if r + 1 < ROWS_PER_W:
            in_copies[1 - cur] = pltpu.async_copy(
                x_hbm.at[base + r + 1], rows[1 - cur], in_sems[1 - cur])
        in_copies[cur].wait()
        if out_copies[cur] is not None:
            out_copies[cur].wait()
        row_ref = rows[cur]
        out_ref = outs[cur]
        last = r == ROWS_PER_W - 1
        spans = ((0, WORDS // 2), (WORDS // 2, WORDS)) if last \
            else ((0, WORDS),)
        tail_copies = []
        for lo_w, hi_w in spans:
            @plsc.parallel_loop(lo_w, hi_w, step=L, unroll=8)
            def _chunk(w, row_ref=row_ref, out_ref=out_ref):
                v = idx_v[pl.ds(w, L)]
                lo = v & 0xFFFF
                hi = v >> 16
                out_ref[pl.ds(2 * w, L)] = plsc.load_gather(row_ref, [lo])
                out_ref[pl.ds(2 * w + L, L)] = plsc.load_gather(row_ref, [hi])

            if last:
                tail_copies.append(pltpu.async_copy(
                    out_ref.at[pl.ds(2 * lo_w, 2 * (hi_w - lo_w))],
                    out_hbm.at[base + r, pl.ds(2 * lo_w, 2 * (hi_w - lo_w))],
                    out_sems[cur]))
            else:
                out_copies[cur] = pltpu.async_copy(
                    out_ref, out_hbm.at[base + r], out_sems[cur])
    out_copies[1 - ((ROWS_PER_W - 1) & 1)].wait()
    for c in tail_copies:
        c.wait()


_gather_call = functools.partial(
    pl.kernel,
    out_type=jax.ShapeDtypeStruct((BATCH, OUTPUTS), jnp.float32),
    mesh=plsc.VectorSubcoreMesh(
        core_axis_name="c", subcore_axis_name="s",
        num_cores=NC, num_subcores=NS,
    ),
    scratch_types=[
        pltpu.VMEM((OUTPUTS // 2,), jnp.int32),
        pltpu.VMEM((INPUTS,), jnp.float32),
        pltpu.VMEM((INPUTS,), jnp.float32),
        pltpu.VMEM((OUTPUTS,), jnp.float32),
        pltpu.VMEM((OUTPUTS,), jnp.float32),
        pltpu.SemaphoreType.DMA,
        pltpu.SemaphoreType.DMA,
        pltpu.SemaphoreType.DMA,
        pltpu.SemaphoreType.DMA,
        pltpu.SemaphoreType.DMA,
    ],
    compiler_params=pltpu.CompilerParams(needs_layout_passes=False, skip_device_barrier=True),
)(_gather_body)


def kernel(x, indices):
    u = indices.astype(jnp.uint32)
    blk = u.reshape(-1, 2, L)
    packed = blk[:, 0, :] | (blk[:, 1, :] << 16)
    idx_words = packed.reshape(-1).astype(jnp.int32)
    return _gather_call(x, idx_words)
